# baseline (speedup 1.0000x reference)
import functools

import jax
import jax.numpy as jnp
from jax import lax
from jax.experimental import pallas as pl
from jax.experimental.pallas import tpu as pltpu
from jax.experimental.pallas import tpu_sc as plsc

_NC = 2
_NS = 16
_L = 16
_NW = _NC * _NS


@jax.jit
def _ultragcn_sc(users, items, table):
    E = users.shape[0]
    D = table.shape[1]
    bpw = E // _NW
    groups = bpw // _L

    mesh = plsc.VectorSubcoreMesh(core_axis_name="c", subcore_axis_name="s")

    @functools.partial(
        pl.kernel,
        out_type=jax.ShapeDtypeStruct((E,), jnp.float32),
        mesh=mesh,
        scratch_types=[
            pltpu.VMEM((bpw,), jnp.int32),
            pltpu.VMEM((bpw,), jnp.int32),
            pltpu.VMEM((bpw, D), jnp.float32),
            pltpu.VMEM((bpw, D), jnp.float32),
            pltpu.VMEM((bpw,), jnp.float32),
            pltpu.SemaphoreType.DMA,
            pltpu.SemaphoreType.DMA,
        ],
        compiler_params=pltpu.CompilerParams(
            use_tc_tiling_on_sc=False, needs_layout_passes=False),
    )
    def k(users_hbm, items_hbm, table_hbm, out_hbm,
          idx_u, idx_i, urows, irows, outv, sem_u, sem_i):
        wid = lax.axis_index("s") * _NC + lax.axis_index("c")
        base = wid * bpw
        pltpu.sync_copy(users_hbm.at[pl.ds(base, bpw)], idx_u)
        pltpu.sync_copy(items_hbm.at[pl.ds(base, bpw)], idx_i)
        cu = pltpu.async_copy(table_hbm.at[idx_u], urows, sem_u)
        ci = pltpu.async_copy(table_hbm.at[idx_i], irows, sem_i)
        cu.wait()
        ci.wait()

        lanes = lax.iota(jnp.int32, _L)

        nchunk = D // _L

        def group_body(g, _):
            e0 = g * _L
            out = jnp.zeros((_L,), jnp.float32)
            for j in range(_L):
                e = e0 + j
                p = jnp.zeros((_L,), jnp.float32)
                for c in range(nchunk):
                    uv = urows[e, pl.ds(c * _L, _L)]
                    iv = irows[e, pl.ds(c * _L, _L)]
                    p = p + uv * iv
                s = jnp.sum(p)
                out = jnp.where(lanes == j, s, out)
            outv[pl.ds(e0, _L)] = out
            return 0

        lax.fori_loop(0, groups, group_body, 0)
        pltpu.sync_copy(outv, out_hbm.at[pl.ds(base, bpw)])

    return k(users, items, table)


def kernel(edge_index, embedding_weight):
    users = edge_index[0]
    items = edge_index[1]
    return _ultragcn_sc(users, items, embedding_weight)
